# baseline (speedup 1.0000x reference)
import functools

import jax
import jax.numpy as jnp
import numpy as np
from jax import lax
from jax.experimental import pallas as pl
from jax.experimental.pallas import tpu as pltpu
from jax.experimental.pallas import tpu_sc as plsc

D_MODEL = 4096
EMB = 64
NTOK = 4 * 8192
NW = 32
COLS = D_MODEL // NW
CHUNK = 256
NCHUNK = NTOK // CHUNK
L = 16

_P = np.zeros((128, 32), np.float32)
for _r in range(64):
    _P[_r, _r // 8] = 0.5
    _P[_r, 8 + _r % 8] = 0.5
    _P[64 + _r, 16 + _r // 8] = 0.5
    _P[64 + _r, 24 + _r % 8] = 0.5


def _table_body(z_ref, w_ref, b_ref, p_ref, t_ref):
    t32 = lax.dot_general(z_ref[...], w_ref[...], (((1,), (1,)), ((), ())),
                          preferred_element_type=jnp.float32)
    t = lax.dot_general(p_ref[...], t32, (((1,), (0,)), ((), ())),
                        preferred_element_type=jnp.float32)
    halfb = (lax.broadcasted_iota(jnp.int32, (128, 1), 0) < 64).astype(jnp.float32)
    t_ref[...] = t + halfb * (0.5 * b_ref[...])


def _build_table(Z, W, b2d, P):
    return pl.pallas_call(
        _table_body,
        out_shape=jax.ShapeDtypeStruct((128, D_MODEL), jnp.float32),
    )(Z, W, b2d, P)


def _sc_body(t_hbm, tf_hbm, out_hbm, t_v, tf_v, o_v):
    wid = lax.axis_index("s") * 2 + lax.axis_index("c")
    col0 = wid * COLS
    pltpu.sync_copy(t_hbm.at[wid], t_v)

    def chunk_body(ci, carry):
        t0 = ci * CHUNK
        pltpu.sync_copy(tf_hbm.at[:, pl.ds(t0, CHUNK)], tf_v)

        zero = jnp.zeros((L,), jnp.int32)
        six = jnp.full((L,), 6, jnp.int32)

        def grp_body(g, c):
            s = g * L
            m = lax.max(lax.min(tf_v[0, pl.ds(s, L)], six), zero)
            w = lax.max(lax.min(tf_v[1, pl.ds(s, L)], six), zero)
            h = lax.max(lax.min(tf_v[2, pl.ds(s, L)], six), zero)
            d = lax.max(lax.min(tf_v[3, pl.ds(s, L)], six), zero)
            b1v = (m * 8 + w) * COLS
            b2v = (h * 8 + d + 64) * COLS
            for k in range(L):
                b1 = b1v[k]
                b2 = b2v[k]
                t = s + k
                for u in range(COLS // L):
                    o_v[t, pl.ds(u * L, L)] = (
                        t_v[pl.ds(b1 + u * L, L)] + t_v[pl.ds(b2 + u * L, L)])
            return c

        lax.fori_loop(0, CHUNK // L, grp_body, 0)
        pltpu.sync_copy(o_v, out_hbm.at[pl.ds(t0, CHUNK), pl.ds(col0, COLS)])
        return carry

    lax.fori_loop(0, NCHUNK, chunk_body, 0)


_sc_gather = functools.partial(
    pl.kernel,
    out_type=jax.ShapeDtypeStruct((NTOK, D_MODEL), jnp.float32),
    mesh=plsc.VectorSubcoreMesh(core_axis_name="c", subcore_axis_name="s"),
    scratch_types=[
        pltpu.VMEM((128 * COLS,), jnp.float32),
        pltpu.VMEM((4, CHUNK), jnp.int32),
        pltpu.VMEM((CHUNK, COLS), jnp.float32),
    ],
)(_sc_body)


def kernel(time_feats, month_w, weekday_w, hour_w, day_w, W, b):
    tf = time_feats.reshape(NTOK, 4).T.astype(jnp.int32)
    wpad = jnp.concatenate([weekday_w, jnp.zeros((1, EMB), jnp.float32)], 0)
    Z = jnp.zeros((32, 4 * EMB), jnp.float32)
    Z = Z.at[0:8, 0:EMB].set(month_w[:8])
    Z = Z.at[8:16, EMB:2 * EMB].set(wpad)
    Z = Z.at[16:24, 2 * EMB:3 * EMB].set(hour_w[:8])
    Z = Z.at[24:32, 3 * EMB:4 * EMB].set(day_w[:8])
    table = _build_table(Z, W, b.reshape(1, D_MODEL), jnp.asarray(_P))
    t2 = table.reshape(128, NW, COLS).transpose(1, 0, 2).reshape(NW, 128 * COLS)
    out = _sc_gather(t2, tf)
    return out.reshape(4, 8192, D_MODEL)


# speedup vs baseline: 1.0063x; 1.0063x over previous
import functools

import jax
import jax.numpy as jnp
import numpy as np
from jax import lax
from jax.experimental import pallas as pl
from jax.experimental.pallas import tpu as pltpu
from jax.experimental.pallas import tpu_sc as plsc

D_MODEL = 4096
EMB = 64
NTOK = 4 * 8192
NW = 32
NBLK = NW // 2
CBLK = D_MODEL // NBLK
PCKW = CBLK // 2
NCOMBO = 4096
CHUNK = 128
NCHUNK = NTOK // CHUNK
L = 16

_P48 = np.zeros((NCOMBO, 32), np.float32)
_r = np.arange(NCOMBO)
_P48[_r, (_r >> 9) & 7] = 0.5
_P48[_r, 8 + ((_r >> 6) & 7)] = 0.5
_P48[_r, 16 + ((_r >> 3) & 7)] = 0.5
_P48[_r, 24 + (_r & 7)] = 0.5

_COLPERM = np.empty(CBLK, np.int32)
for _g in range(CBLK // 32):
    for _i in range(16):
        _COLPERM[_g * 32 + 2 * _i] = _g * 32 + _i
        _COLPERM[_g * 32 + 2 * _i + 1] = _g * 32 + 16 + _i


def _table_body(p48_ref, z_ref, w_ref, b_ref, t_ref):
    t32 = lax.dot_general(z_ref[...], w_ref[...], (((1,), (1,)), ((), ())),
                          preferred_element_type=jnp.float32)
    t4 = lax.dot_general(p48_ref[...], t32, (((1,), (0,)), ((), ())),
                         preferred_element_type=jnp.float32)
    t_ref[...] = (t4 + 0.5 * b_ref[...]).astype(jnp.bfloat16)


def _build_table(P48, Z, W, b2d):
    return pl.pallas_call(
        _table_body,
        grid=(NBLK,),
        in_specs=[
            pl.BlockSpec((NCOMBO, 32), lambda w: (0, 0)),
            pl.BlockSpec((32, 4 * EMB), lambda w: (0, 0)),
            pl.BlockSpec((CBLK, 4 * EMB), lambda w: (w, 0)),
            pl.BlockSpec((1, CBLK), lambda w: (0, w)),
        ],
        out_specs=pl.BlockSpec((NCOMBO, CBLK), lambda w: (w, 0)),
        out_shape=jax.ShapeDtypeStruct((NBLK * NCOMBO, CBLK), jnp.bfloat16),
    )(P48, Z, W, b2d)


def _sc_body(t_hbm, tf_hbm, out_hbm, tf_v, p_v, r_v, o_v, gsem, out_sem):
    wid = lax.axis_index("s") * 2 + lax.axis_index("c")
    blk = wid // 2
    par = wid % 2
    col0 = blk * CBLK
    row0 = blk * NCOMBO
    zero = jnp.zeros((L,), jnp.int32)
    six = jnp.full((L,), 6, jnp.int32)
    nloc = NCHUNK // 2

    def prep(j, nb):
        t0 = (2 * j + par) * CHUNK
        pltpu.sync_copy(tf_hbm.at[:, pl.ds(t0, CHUNK)], tf_v)

        def idx_body(g, c):
            s = g * L
            m = lax.max(lax.min(tf_v[0, pl.ds(s, L)], six), zero)
            w = lax.max(lax.min(tf_v[1, pl.ds(s, L)], six), zero)
            h = lax.max(lax.min(tf_v[2, pl.ds(s, L)], six), zero)
            d = lax.max(lax.min(tf_v[3, pl.ds(s, L)], six), zero)
            p_v[nb, pl.ds(s, L)] = (m * 512 + w * 64 + h * 8 + d) + row0
            return c

        lax.fori_loop(0, CHUNK // L, idx_body, 0)
        pltpu.async_copy(t_hbm.at[p_v.at[nb]], r_v.at[nb], gsem)

    prep(0, 0)
    himask = jnp.full((L,), -65536, jnp.int32)
    sh16 = jnp.full((L,), 16, jnp.int32)

    def pair_body(i, carry):
        for b in range(2):
            j = 2 * i + b
            t0 = (2 * j + par) * CHUNK
            nb = 1 - b
            nj = lax.rem(j + 1, nloc)
            pltpu.make_async_copy(t_hbm.at[p_v.at[b]], r_v.at[b], gsem).wait()
            @pl.when(j >= 1)
            def _():
                pltpu.make_async_copy(
                    o_v.at[nb],
                    out_hbm.at[pl.ds(t0, CHUNK), pl.ds(col0, CBLK)],
                    out_sem).wait()

            prep(nj, nb)

            def tok_body(t, c):
                for g in range(PCKW // L):
                    v = r_v[b, t, pl.ds(g * L, L)]
                    o_v[b, t, pl.ds(g * 2 * L, L)] = lax.shift_left(v, sh16)
                    o_v[b, t, pl.ds(g * 2 * L + L, L)] = lax.bitwise_and(
                        v, himask)
                return c

            lax.fori_loop(0, CHUNK, tok_body, 0)
            pltpu.async_copy(
                o_v.at[b], out_hbm.at[pl.ds(t0, CHUNK), pl.ds(col0, CBLK)],
                out_sem)
        return carry

    lax.fori_loop(0, nloc // 2, pair_body, 0)
    pltpu.make_async_copy(
        o_v.at[1], out_hbm.at[pl.ds(0, CHUNK), pl.ds(col0, CBLK)],
        out_sem).wait()
    pltpu.make_async_copy(t_hbm.at[p_v.at[0]], r_v.at[0], gsem).wait()


_sc_gather = functools.partial(
    pl.kernel,
    out_type=jax.ShapeDtypeStruct((NTOK, D_MODEL), jnp.int32),
    mesh=plsc.VectorSubcoreMesh(core_axis_name="c", subcore_axis_name="s"),
    scratch_types=[
        pltpu.VMEM((4, CHUNK), jnp.int32),
        pltpu.VMEM((2, CHUNK), jnp.int32),
        pltpu.VMEM((2, CHUNK, PCKW), jnp.int32),
        pltpu.VMEM((2, CHUNK, CBLK), jnp.int32),
        pltpu.SemaphoreType.DMA,
        pltpu.SemaphoreType.DMA,
    ],
)(_sc_body)


def kernel(time_feats, month_w, weekday_w, hour_w, day_w, W, b):
    tf = time_feats.reshape(NTOK, 4).T.astype(jnp.int32)
    wpad = jnp.concatenate([weekday_w, jnp.zeros((1, EMB), jnp.float32)], 0)
    Z = jnp.zeros((32, 4 * EMB), jnp.float32)
    Z = Z.at[0:8, 0:EMB].set(month_w[:8])
    Z = Z.at[8:16, EMB:2 * EMB].set(wpad)
    Z = Z.at[16:24, 2 * EMB:3 * EMB].set(hour_w[:8])
    Z = Z.at[24:32, 3 * EMB:4 * EMB].set(day_w[:8])
    t4 = _build_table(jnp.asarray(_P48), Z, W, b.reshape(1, D_MODEL))
    t4p = lax.bitcast_convert_type(
        t4[:, jnp.asarray(_COLPERM)].reshape(NBLK * NCOMBO, PCKW, 2),
        jnp.int32)
    out = _sc_gather(t4p, tf)
    return lax.bitcast_convert_type(out, jnp.float32).reshape(
        4, 8192, D_MODEL)


# speedup vs baseline: 1.8378x; 1.8262x over previous
import functools

import jax
import jax.numpy as jnp
import numpy as np
from jax import lax
from jax.experimental import pallas as pl
from jax.experimental.pallas import tpu as pltpu
from jax.experimental.pallas import tpu_sc as plsc

D_MODEL = 4096
EMB = 64
NTOK = 4 * 8192
NW = 32
COLS = D_MODEL // NW
CHUNK = 128
NCHUNK = NTOK // CHUNK
L = 16

_P = np.zeros((128, 32), np.float32)
for _r in range(64):
    _P[_r, _r // 8] = 0.5
    _P[_r, 8 + _r % 8] = 0.5
    _P[64 + _r, 16 + _r // 8] = 0.5
    _P[64 + _r, 24 + _r % 8] = 0.5


def _table_body(z_ref, w_ref, b_ref, p_ref, t_ref):
    t32 = lax.dot_general(z_ref[...], w_ref[...], (((1,), (1,)), ((), ())),
                          preferred_element_type=jnp.float32)
    t = lax.dot_general(p_ref[...], t32, (((1,), (0,)), ((), ())),
                        preferred_element_type=jnp.float32)
    halfb = (lax.broadcasted_iota(jnp.int32, (128, 1), 0) < 64).astype(jnp.float32)
    t_ref[...] = t + halfb * (0.5 * b_ref[...])


def _build_table(Z, W, b2d, P):
    return pl.pallas_call(
        _table_body,
        out_shape=jax.ShapeDtypeStruct((128, D_MODEL), jnp.float32),
    )(Z, W, b2d, P)


def _sc_body(t_hbm, tf_hbm, out_hbm, tf_v, p1_v, p2_v, r1_v, r2_v, gsem,
             out_sem):
    wid = lax.axis_index("s") * 2 + lax.axis_index("c")
    col0 = wid * COLS
    row0 = wid * 128
    zero = jnp.zeros((L,), jnp.int32)
    six = jnp.full((L,), 6, jnp.int32)

    def prep(ci, nb):
        t0 = ci * CHUNK
        pltpu.sync_copy(tf_hbm.at[:, pl.ds(t0, CHUNK)], tf_v)

        def idx_body(g, c):
            s = g * L
            m = lax.max(lax.min(tf_v[0, pl.ds(s, L)], six), zero)
            w = lax.max(lax.min(tf_v[1, pl.ds(s, L)], six), zero)
            h = lax.max(lax.min(tf_v[2, pl.ds(s, L)], six), zero)
            d = lax.max(lax.min(tf_v[3, pl.ds(s, L)], six), zero)
            p1_v[nb, pl.ds(s, L)] = m * 8 + w + row0
            p2_v[nb, pl.ds(s, L)] = h * 8 + d + 64 + row0
            return c

        lax.fori_loop(0, CHUNK // L, idx_body, 0)
        pltpu.async_copy(t_hbm.at[p1_v.at[nb]], r1_v.at[nb], gsem)
        pltpu.async_copy(t_hbm.at[p2_v.at[nb]], r2_v.at[nb], gsem)

    prep(0, 0)

    def pair_body(i, carry):
        for b in range(2):
            ci = 2 * i + b
            t0 = ci * CHUNK
            nb = 1 - b
            nci = lax.rem(ci + 1, NCHUNK)
            pltpu.make_async_copy(t_hbm.at[p1_v.at[b]], r1_v.at[b],
                                  gsem).wait()
            pltpu.make_async_copy(t_hbm.at[p2_v.at[b]], r2_v.at[b],
                                  gsem).wait()
            @pl.when(ci >= 1)
            def _():
                pltpu.make_async_copy(
                    r1_v.at[nb],
                    out_hbm.at[pl.ds(t0, CHUNK), pl.ds(col0, COLS)],
                    out_sem).wait()

            prep(nci, nb)

            def add_body(t, c):
                for u in range(COLS // L):
                    r1_v[b, t, pl.ds(u * L, L)] = (
                        r1_v[b, t, pl.ds(u * L, L)]
                        + r2_v[b, t, pl.ds(u * L, L)])
                return c

            lax.fori_loop(0, CHUNK, add_body, 0)
            pltpu.async_copy(
                r1_v.at[b], out_hbm.at[pl.ds(t0, CHUNK), pl.ds(col0, COLS)],
                out_sem)
        return carry

    lax.fori_loop(0, NCHUNK // 2, pair_body, 0)
    pltpu.make_async_copy(
        r1_v.at[1], out_hbm.at[pl.ds(0, CHUNK), pl.ds(col0, COLS)],
        out_sem).wait()
    pltpu.make_async_copy(t_hbm.at[p1_v.at[0]], r1_v.at[0], gsem).wait()
    pltpu.make_async_copy(t_hbm.at[p2_v.at[0]], r2_v.at[0], gsem).wait()


_sc_gather = functools.partial(
    pl.kernel,
    out_type=jax.ShapeDtypeStruct((NTOK, D_MODEL), jnp.float32),
    mesh=plsc.VectorSubcoreMesh(core_axis_name="c", subcore_axis_name="s"),
    scratch_types=[
        pltpu.VMEM((4, CHUNK), jnp.int32),
        pltpu.VMEM((2, CHUNK), jnp.int32),
        pltpu.VMEM((2, CHUNK), jnp.int32),
        pltpu.VMEM((2, CHUNK, COLS), jnp.float32),
        pltpu.VMEM((2, CHUNK, COLS), jnp.float32),
        pltpu.SemaphoreType.DMA,
        pltpu.SemaphoreType.DMA,
    ],
)(_sc_body)


def kernel(time_feats, month_w, weekday_w, hour_w, day_w, W, b):
    tf = time_feats.reshape(NTOK, 4).T.astype(jnp.int32)
    wpad = jnp.concatenate([weekday_w, jnp.zeros((1, EMB), jnp.float32)], 0)
    Z = jnp.zeros((32, 4 * EMB), jnp.float32)
    Z = Z.at[0:8, 0:EMB].set(month_w[:8])
    Z = Z.at[8:16, EMB:2 * EMB].set(wpad)
    Z = Z.at[16:24, 2 * EMB:3 * EMB].set(hour_w[:8])
    Z = Z.at[24:32, 3 * EMB:4 * EMB].set(day_w[:8])
    table = _build_table(Z, W, b.reshape(1, D_MODEL), jnp.asarray(_P))
    t2 = table.reshape(128, NW, COLS).transpose(1, 0, 2).reshape(NW * 128, COLS)
    out = _sc_gather(t2, tf)
    return out.reshape(4, 8192, D_MODEL)


# speedup vs baseline: 2.4007x; 1.3063x over previous
import functools

import jax
import jax.numpy as jnp
import numpy as np
from jax import lax
from jax.experimental import pallas as pl
from jax.experimental.pallas import tpu as pltpu
from jax.experimental.pallas import tpu_sc as plsc

D_MODEL = 4096
EMB = 64
NTOK = 4 * 8192
NW = 32
COLS = D_MODEL // NW
CHUNK = 128
NCHUNK = NTOK // CHUNK
L = 16

_P = np.zeros((128, 32), np.float32)
for _r in range(64):
    _P[_r, _r // 8] = 0.5
    _P[_r, 8 + _r % 8] = 0.5
    _P[64 + _r, 16 + _r // 8] = 0.5
    _P[64 + _r, 24 + _r % 8] = 0.5


def _table_body(z_ref, w_ref, b_ref, p_ref, t_ref):
    t32 = lax.dot_general(z_ref[...], w_ref[...], (((1,), (1,)), ((), ())),
                          preferred_element_type=jnp.float32)
    t = lax.dot_general(p_ref[...], t32, (((1,), (0,)), ((), ())),
                        preferred_element_type=jnp.float32)
    halfb = (lax.broadcasted_iota(jnp.int32, (128, 1), 0) < 64).astype(jnp.float32)
    t_ref[...] = t + halfb * (0.5 * b_ref[...])


def _build_table(Z, W, b2d, P):
    return pl.pallas_call(
        _table_body,
        out_shape=jax.ShapeDtypeStruct((128, D_MODEL), jnp.float32),
    )(Z, W, b2d, P)


def _sc_body(t_hbm, tf_hbm, out_hbm, t_v, tf_v, p1_v, p2_v, r1_v, r2_v, gsem,
             out_sem):
    wid = lax.axis_index("s") * 2 + lax.axis_index("c")
    col0 = wid * COLS
    zero = jnp.zeros((L,), jnp.int32)
    six = jnp.full((L,), 6, jnp.int32)
    row0 = wid * 128
    @pl.when(lax.axis_index("s") == 0)
    def _():
        pltpu.sync_copy(t_hbm, t_v)
    plsc.subcore_barrier()

    def prep(ci, nb):
        t0 = ci * CHUNK
        pltpu.sync_copy(tf_hbm.at[:, pl.ds(t0, CHUNK)], tf_v)

        def idx_body(g, c):
            s = g * L
            m = lax.max(lax.min(tf_v[0, pl.ds(s, L)], six), zero)
            w = lax.max(lax.min(tf_v[1, pl.ds(s, L)], six), zero)
            h = lax.max(lax.min(tf_v[2, pl.ds(s, L)], six), zero)
            d = lax.max(lax.min(tf_v[3, pl.ds(s, L)], six), zero)
            p1_v[nb, pl.ds(s, L)] = m * 8 + w + row0
            p2_v[nb, pl.ds(s, L)] = h * 8 + d + 64 + row0
            return c

        lax.fori_loop(0, CHUNK // L, idx_body, 0)
        pltpu.async_copy(t_v.at[p1_v.at[nb]], r1_v.at[nb], gsem)
        pltpu.async_copy(t_v.at[p2_v.at[nb]], r2_v.at[nb], gsem)

    prep(0, 0)

    def pair_body(i, carry):
        for b in range(2):
            ci = 2 * i + b
            t0 = ci * CHUNK
            nb = 1 - b
            nci = lax.rem(ci + 1, NCHUNK)
            pltpu.make_async_copy(t_v.at[p1_v.at[b]], r1_v.at[b],
                                  gsem).wait()
            pltpu.make_async_copy(t_v.at[p2_v.at[b]], r2_v.at[b],
                                  gsem).wait()
            @pl.when(ci >= 1)
            def _():
                pltpu.make_async_copy(
                    r1_v.at[nb],
                    out_hbm.at[pl.ds(t0, CHUNK), pl.ds(col0, COLS)],
                    out_sem).wait()

            prep(nci, nb)

            def add_body(t, c):
                for u in range(COLS // L):
                    r1_v[b, t, pl.ds(u * L, L)] = (
                        r1_v[b, t, pl.ds(u * L, L)]
                        + r2_v[b, t, pl.ds(u * L, L)])
                return c

            lax.fori_loop(0, CHUNK, add_body, 0)
            pltpu.async_copy(
                r1_v.at[b], out_hbm.at[pl.ds(t0, CHUNK), pl.ds(col0, COLS)],
                out_sem)
        return carry

    lax.fori_loop(0, NCHUNK // 2, pair_body, 0)
    pltpu.make_async_copy(
        r1_v.at[1], out_hbm.at[pl.ds(0, CHUNK), pl.ds(col0, COLS)],
        out_sem).wait()
    pltpu.make_async_copy(t_v.at[p1_v.at[0]], r1_v.at[0], gsem).wait()
    pltpu.make_async_copy(t_v.at[p2_v.at[0]], r2_v.at[0], gsem).wait()


_sc_gather = functools.partial(
    pl.kernel,
    out_type=jax.ShapeDtypeStruct((NTOK, D_MODEL), jnp.float32),
    mesh=plsc.VectorSubcoreMesh(core_axis_name="c", subcore_axis_name="s"),
    scratch_types=[
        pltpu.VMEM_SHARED((NW * 128, COLS), jnp.float32),
        pltpu.VMEM((4, CHUNK), jnp.int32),
        pltpu.VMEM((2, CHUNK), jnp.int32),
        pltpu.VMEM((2, CHUNK), jnp.int32),
        pltpu.VMEM((2, CHUNK, COLS), jnp.float32),
        pltpu.VMEM((2, CHUNK, COLS), jnp.float32),
        pltpu.SemaphoreType.DMA,
        pltpu.SemaphoreType.DMA,
    ],
)(_sc_body)


def kernel(time_feats, month_w, weekday_w, hour_w, day_w, W, b):
    tf = time_feats.reshape(NTOK, 4).T.astype(jnp.int32)
    wpad = jnp.concatenate([weekday_w, jnp.zeros((1, EMB), jnp.float32)], 0)
    Z = jnp.zeros((32, 4 * EMB), jnp.float32)
    Z = Z.at[0:8, 0:EMB].set(month_w[:8])
    Z = Z.at[8:16, EMB:2 * EMB].set(wpad)
    Z = Z.at[16:24, 2 * EMB:3 * EMB].set(hour_w[:8])
    Z = Z.at[24:32, 3 * EMB:4 * EMB].set(day_w[:8])
    table = _build_table(Z, W, b.reshape(1, D_MODEL), jnp.asarray(_P))
    t2 = table.reshape(128, NW, COLS).transpose(1, 0, 2).reshape(NW * 128, COLS)
    out = _sc_gather(t2, tf)
    return out.reshape(4, 8192, D_MODEL)


# speedup vs baseline: 3.1904x; 1.3290x over previous
import functools

import jax
import jax.numpy as jnp
import numpy as np
from jax import lax
from jax.experimental import pallas as pl
from jax.experimental.pallas import tpu as pltpu
from jax.experimental.pallas import tpu_sc as plsc

D_MODEL = 4096
EMB = 64
NTOK = 4 * 8192
NW = 32
COLS = D_MODEL // NW
CHUNK = 128
NCHUNK = NTOK // CHUNK
L = 16

_P = np.zeros((128, 32), np.float32)
for _r in range(64):
    _P[_r, _r // 8] = 0.5
    _P[_r, 8 + _r % 8] = 0.5
    _P[64 + _r, 16 + _r // 8] = 0.5
    _P[64 + _r, 24 + _r % 8] = 0.5


def _table_body(z_ref, w_ref, b_ref, p_ref, t_ref):
    t32 = lax.dot_general(z_ref[...], w_ref[...], (((1,), (1,)), ((), ())),
                          preferred_element_type=jnp.float32)
    t = lax.dot_general(p_ref[...], t32, (((1,), (0,)), ((), ())),
                        preferred_element_type=jnp.float32)
    halfb = (lax.broadcasted_iota(jnp.int32, (128, 1), 0) < 64).astype(jnp.float32)
    t_ref[...] = t + halfb * (0.5 * b_ref[...])


def _build_table(Z, W, b2d, P):
    return pl.pallas_call(
        _table_body,
        out_shape=jax.ShapeDtypeStruct((128, D_MODEL), jnp.float32),
    )(Z, W, b2d, P)


def _sc_body(t_hbm, tf_hbm, out_hbm, t_v, tf_v, p1_v, p2_v, r1_v, r2_v, gsem,
             out_sem, tf_sem):
    wid = lax.axis_index("s") * 2 + lax.axis_index("c")
    col0 = wid * COLS
    zero = jnp.zeros((L,), jnp.int32)
    six = jnp.full((L,), 6, jnp.int32)
    row0 = wid * 128
    @pl.when(lax.axis_index("s") == 0)
    def _():
        pltpu.sync_copy(t_hbm, t_v)
    plsc.subcore_barrier()

    def prep(ci, nb):
        pltpu.make_async_copy(tf_hbm.at[:, pl.ds(0, CHUNK)], tf_v.at[nb],
                              tf_sem).wait()

        def idx_body(g, c):
            s = g * L
            m = lax.max(lax.min(tf_v[nb, 0, pl.ds(s, L)], six), zero)
            w = lax.max(lax.min(tf_v[nb, 1, pl.ds(s, L)], six), zero)
            h = lax.max(lax.min(tf_v[nb, 2, pl.ds(s, L)], six), zero)
            d = lax.max(lax.min(tf_v[nb, 3, pl.ds(s, L)], six), zero)
            p1_v[nb, pl.ds(s, L)] = m * 8 + w + row0
            p2_v[nb, pl.ds(s, L)] = h * 8 + d + 64 + row0
            return c

        lax.fori_loop(0, CHUNK // L, idx_body, 0)
        pltpu.async_copy(t_v.at[p1_v.at[nb]], r1_v.at[nb], gsem)
        pltpu.async_copy(t_v.at[p2_v.at[nb]], r2_v.at[nb], gsem)
        nt0 = lax.rem(ci + 1, NCHUNK) * CHUNK
        pltpu.async_copy(tf_hbm.at[:, pl.ds(nt0, CHUNK)], tf_v.at[1 - nb],
                         tf_sem)

    pltpu.async_copy(tf_hbm.at[:, pl.ds(0, CHUNK)], tf_v.at[0], tf_sem)
    prep(0, 0)

    def pair_body(i, carry):
        for b in range(2):
            ci = 2 * i + b
            t0 = ci * CHUNK
            nb = 1 - b
            nci = lax.rem(ci + 1, NCHUNK)
            pltpu.make_async_copy(t_v.at[p1_v.at[b]], r1_v.at[b],
                                  gsem).wait()
            pltpu.make_async_copy(t_v.at[p2_v.at[b]], r2_v.at[b],
                                  gsem).wait()
            @pl.when(ci >= 1)
            def _():
                pltpu.make_async_copy(
                    r1_v.at[nb],
                    out_hbm.at[pl.ds(t0, CHUNK), pl.ds(col0, COLS)],
                    out_sem).wait()

            prep(nci, nb)

            def add_body(t, c):
                for u in range(COLS // L):
                    r1_v[b, t, pl.ds(u * L, L)] = (
                        r1_v[b, t, pl.ds(u * L, L)]
                        + r2_v[b, t, pl.ds(u * L, L)])
                return c

            lax.fori_loop(0, CHUNK, add_body, 0)
            pltpu.async_copy(
                r1_v.at[b], out_hbm.at[pl.ds(t0, CHUNK), pl.ds(col0, COLS)],
                out_sem)
        return carry

    lax.fori_loop(0, NCHUNK // 2, pair_body, 0)
    pltpu.make_async_copy(
        r1_v.at[1], out_hbm.at[pl.ds(0, CHUNK), pl.ds(col0, COLS)],
        out_sem).wait()
    pltpu.make_async_copy(t_v.at[p1_v.at[0]], r1_v.at[0], gsem).wait()
    pltpu.make_async_copy(t_v.at[p2_v.at[0]], r2_v.at[0], gsem).wait()
    pltpu.make_async_copy(tf_hbm.at[:, pl.ds(0, CHUNK)], tf_v.at[1],
                          tf_sem).wait()


_sc_gather = functools.partial(
    pl.kernel,
    out_type=jax.ShapeDtypeStruct((NTOK, D_MODEL), jnp.float32),
    mesh=plsc.VectorSubcoreMesh(core_axis_name="c", subcore_axis_name="s"),
    scratch_types=[
        pltpu.VMEM_SHARED((NW * 128, COLS), jnp.float32),
        pltpu.VMEM((2, 4, CHUNK), jnp.int32),
        pltpu.VMEM((2, CHUNK), jnp.int32),
        pltpu.VMEM((2, CHUNK), jnp.int32),
        pltpu.VMEM((2, CHUNK, COLS), jnp.float32),
        pltpu.VMEM((2, CHUNK, COLS), jnp.float32),
        pltpu.SemaphoreType.DMA,
        pltpu.SemaphoreType.DMA,
        pltpu.SemaphoreType.DMA,
    ],
)(_sc_body)


def kernel(time_feats, month_w, weekday_w, hour_w, day_w, W, b):
    tf = time_feats.reshape(NTOK, 4).T.astype(jnp.int32)
    wpad = jnp.concatenate([weekday_w, jnp.zeros((1, EMB), jnp.float32)], 0)
    Z = jnp.zeros((32, 4 * EMB), jnp.float32)
    Z = Z.at[0:8, 0:EMB].set(month_w[:8])
    Z = Z.at[8:16, EMB:2 * EMB].set(wpad)
    Z = Z.at[16:24, 2 * EMB:3 * EMB].set(hour_w[:8])
    Z = Z.at[24:32, 3 * EMB:4 * EMB].set(day_w[:8])
    table = _build_table(Z, W, b.reshape(1, D_MODEL), jnp.asarray(_P))
    t2 = table.reshape(128, NW, COLS).transpose(1, 0, 2).reshape(NW * 128, COLS)
    out = _sc_gather(t2, tf)
    return out.reshape(4, 8192, D_MODEL)
